# baseline (speedup 1.0000x reference)
import functools

import jax
import jax.numpy as jnp
from jax import lax
from jax.experimental import pallas as pl
from jax.experimental.pallas import tpu as pltpu
from jax.experimental.pallas import tpu_sc as plsc

N = 10000
D = 128
N_PAD = 10240
E = 320000
NC = 2
NS = 16
NW = NC * NS
CH = 128
K = 2 * (-(-E // (NW * CH * 2)))
E_PAD = NW * CH * K
NCHUNK = E_PAD // CH
KA = 160
KB = NCHUNK // NS - KA
HB = 16
ROWS_PER_TILE = N_PAD // NS
BT = 512

_mesh = plsc.VectorSubcoreMesh(core_axis_name="c", subcore_axis_name="s")


@functools.partial(
    pl.kernel,
    out_type=(
        jax.ShapeDtypeStruct((NC, N_PAD, D), jnp.float32),
        jax.ShapeDtypeStruct((NC, N_PAD, D), jnp.float32),
    ),
    mesh=_mesh,
    scratch_types=[
        pltpu.VMEM((K, CH), jnp.int32),
        pltpu.VMEM((CH, D), jnp.float32),
        pltpu.VMEM((CH, D), jnp.float32),
        pltpu.VMEM_SHARED((N_PAD, D), jnp.float32),
    ],
)
def _sc_degrees(ones_hbm, src_hbm, dst_hbm, out_s, out_d, idx_v, ones_v,
                buf_v, deg_sh):
    c = lax.axis_index("c")
    s = lax.axis_index("s")
    pltpu.sync_copy(ones_hbm.at[0], ones_v)
    w = s * NC + c

    for idx_hbm, out_hbm in ((src_hbm, out_s), (dst_hbm, out_d)):
        pltpu.sync_copy(ones_hbm.at[1], buf_v)
        pltpu.sync_copy(idx_hbm.at[pl.ds(w * K, K)], idx_v)
        for t in range(ROWS_PER_TILE // CH):
            r0 = s * ROWS_PER_TILE + t * CH
            pltpu.sync_copy(buf_v, deg_sh.at[pl.ds(r0, CH)])
        plsc.subcore_barrier()

        def body(j, carry):
            pltpu.sync_copy(ones_v, deg_sh.at[idx_v.at[j]], add=True)
            return carry

        lax.fori_loop(0, K, body, 0)
        plsc.subcore_barrier()
        for t in range(ROWS_PER_TILE // CH):
            r0 = s * ROWS_PER_TILE + t * CH
            pltpu.sync_copy(deg_sh.at[pl.ds(r0, CH)], buf_v)
            pltpu.sync_copy(buf_v, out_hbm.at[c, pl.ds(r0, CH)])
        plsc.subcore_barrier()


@functools.partial(
    pl.kernel,
    out_type=jax.ShapeDtypeStruct((N_PAD, D), jnp.float32),
    mesh=_mesh,
    scratch_types=[
        pltpu.VMEM((HB, CH), jnp.int32),
        pltpu.VMEM((HB, CH), jnp.int32),
        pltpu.VMEM((CH, D), jnp.float32),
        pltpu.VMEM((CH, D), jnp.float32),
        pltpu.VMEM_SHARED((N_PAD, D), jnp.float32),
        pltpu.SemaphoreType.DMA,
        pltpu.SemaphoreType.DMA,
    ],
)
def _sc_aggregate(h_hbm, src_hbm, dst_hbm, out_hbm, src_v, dst_v, rows0,
                  rows1, agg_sh, sem0, sem1):
    c = lax.axis_index("c")
    s = lax.axis_index("s")

    @pl.when(c == 0)
    def _():
        zero16 = jnp.zeros((16,), jnp.float32)

        def zrow(i, carry):
            for j in range(D // 16):
                rows0[i, pl.ds(j * 16, 16)] = zero16
            return carry

        lax.fori_loop(0, CH, zrow, 0)
        for t in range(ROWS_PER_TILE // CH):
            pltpu.sync_copy(rows0,
                            agg_sh.at[pl.ds(s * ROWS_PER_TILE + t * CH, CH)])
        plsc.subcore_barrier()

        start = s * KA

        def block(bi, carry):
            b0 = start + bi * HB
            pltpu.sync_copy(src_hbm.at[pl.ds(b0, HB)], src_v)
            pltpu.sync_copy(dst_hbm.at[pl.ds(b0, HB)], dst_v)
            pltpu.async_copy(h_hbm.at[src_v.at[0]], rows0, sem0)

            def body(jj, carry2):
                a = 2 * jj
                pltpu.async_copy(h_hbm.at[src_v.at[a + 1]], rows1, sem1)
                pltpu.make_async_copy(h_hbm.at[src_v.at[a]], rows0, sem0).wait()
                pltpu.sync_copy(rows0, agg_sh.at[dst_v.at[a]], add=True)

                @pl.when(jj < HB // 2 - 1)
                def _():
                    pltpu.async_copy(h_hbm.at[src_v.at[a + 2]], rows0, sem0)

                pltpu.make_async_copy(h_hbm.at[src_v.at[a + 1]], rows1,
                                      sem1).wait()
                pltpu.sync_copy(rows1, agg_sh.at[dst_v.at[a + 1]], add=True)
                return carry2

            lax.fori_loop(0, HB // 2, body, 0)
            return carry

        lax.fori_loop(0, KA // HB, block, 0)
        plsc.subcore_barrier()
        for t in range(ROWS_PER_TILE // CH):
            r0 = s * ROWS_PER_TILE + t * CH
            pltpu.sync_copy(agg_sh.at[pl.ds(r0, CH)], rows0)
            pltpu.sync_copy(rows0, out_hbm.at[pl.ds(r0, CH)])


def _tc_pre_body(deg_s_ref, deg_d_ref, x_ref, rsout_ref, rsin_ref, h0_ref):
    ds_ = deg_s_ref[0] + deg_s_ref[1]
    dd_ = deg_d_ref[0] + deg_d_ref[1]
    so = lax.rsqrt(jnp.maximum(ds_[:, :1], 1.0))
    si = lax.rsqrt(jnp.maximum(dd_[:, :1], 1.0))
    rsout = jnp.broadcast_to(so, (BT, D))
    rsin = jnp.broadcast_to(si, (BT, D))
    rsout_ref[...] = rsout
    rsin_ref[...] = rsin
    h0_ref[...] = x_ref[...] * rsout


def _tc_pre(deg_s, deg_d, x):
    grid = (N_PAD // BT,)
    return pl.pallas_call(
        _tc_pre_body,
        grid=grid,
        in_specs=[
            pl.BlockSpec((NC, BT, D), lambda i: (0, i, 0)),
            pl.BlockSpec((NC, BT, D), lambda i: (0, i, 0)),
            pl.BlockSpec((BT, D), lambda i: (i, 0)),
        ],
        out_specs=[
            pl.BlockSpec((BT, D), lambda i: (i, 0)),
            pl.BlockSpec((BT, D), lambda i: (i, 0)),
            pl.BlockSpec((BT, D), lambda i: (i, 0)),
        ],
        out_shape=[
            jax.ShapeDtypeStruct((N_PAD, D), jnp.float32),
            jax.ShapeDtypeStruct((N_PAD, D), jnp.float32),
            jax.ShapeDtypeStruct((N_PAD, D), jnp.float32),
        ],
    )(deg_s, deg_d, x)


def _tc_mid_body(agg_ref, rsin_ref, rsout_ref, w_ref, b_ref, out_ref):
    a = agg_ref[...] * rsin_ref[...]
    h = jnp.dot(a, w_ref[...], preferred_element_type=jnp.float32) + b_ref[...]
    out_ref[...] = jnp.maximum(h, 0.0) * rsout_ref[...]


def _tc_mid(agg, rsin, rsout, w, b):
    grid = (N_PAD // BT,)
    return pl.pallas_call(
        _tc_mid_body,
        grid=grid,
        in_specs=[
            pl.BlockSpec((BT, D), lambda i: (i, 0)),
            pl.BlockSpec((BT, D), lambda i: (i, 0)),
            pl.BlockSpec((BT, D), lambda i: (i, 0)),
            pl.BlockSpec((D, D), lambda i: (0, 0)),
            pl.BlockSpec((1, D), lambda i: (0, 0)),
        ],
        out_specs=pl.BlockSpec((BT, D), lambda i: (i, 0)),
        out_shape=jax.ShapeDtypeStruct((N_PAD, D), jnp.float32),
    )(agg, rsin, rsout, w, b)


def _tc_fin_body(agg_ref, rsin_ref, w_ref, b_ref, out_ref):
    i = pl.program_id(0)
    a = agg_ref[...] * rsin_ref[...]
    h = jnp.dot(a, w_ref[...], preferred_element_type=jnp.float32) + b_ref[...]
    h = jnp.maximum(h, 0.0)
    rows = lax.broadcasted_iota(jnp.int32, (BT, D), 0) + i * BT
    h = jnp.where(rows < N, h, 0.0)
    part = jnp.sum(h, axis=0, keepdims=True) * (1.0 / N)

    @pl.when(i == 0)
    def _():
        out_ref[...] = part

    @pl.when(i > 0)
    def _():
        out_ref[...] = out_ref[...] + part


def _tc_fin(agg, rsin, w, b):
    grid = (N_PAD // BT,)
    return pl.pallas_call(
        _tc_fin_body,
        grid=grid,
        in_specs=[
            pl.BlockSpec((BT, D), lambda i: (i, 0)),
            pl.BlockSpec((BT, D), lambda i: (i, 0)),
            pl.BlockSpec((D, D), lambda i: (0, 0)),
            pl.BlockSpec((1, D), lambda i: (0, 0)),
        ],
        out_specs=pl.BlockSpec((1, D), lambda i: (0, 0)),
        out_shape=jax.ShapeDtypeStruct((1, D), jnp.float32),
    )(agg, rsin, w, b)


def kernel(n_feat, edge_index, W1, b1, W2, b2, W3, b3):
    src = edge_index[0].astype(jnp.int32)
    dst = edge_index[1].astype(jnp.int32)
    pad = jnp.full((E_PAD - E,), N, jnp.int32)
    src_p = jnp.concatenate([src, pad]).reshape(NCHUNK, CH)
    dst_p = jnp.concatenate([dst, pad]).reshape(NCHUNK, CH)
    x_p = jnp.pad(n_feat, ((0, N_PAD - N), (0, 0)))

    ones_c = jnp.stack([jnp.ones((CH, D), jnp.float32),
                        jnp.zeros((CH, D), jnp.float32)])
    deg_s, deg_d = _sc_degrees(ones_c, src_p, dst_p)
    rsout, rsin, h = _tc_pre(deg_s, deg_d, x_p)
    for w, b in ((W1, b1), (W2, b2)):
        agg = _sc_aggregate(h, src_p, dst_p)
        h = _tc_mid(agg, rsin, rsout, w, b.reshape(1, D))
    agg = _sc_aggregate(h, src_p, dst_p)
    return _tc_fin(agg, rsin, W3, b3.reshape(1, D))


# speedup vs baseline: 3.4199x; 3.4199x over previous
import functools

import jax
import jax.numpy as jnp
from jax import lax
from jax.experimental import pallas as pl
from jax.experimental.pallas import tpu as pltpu
from jax.experimental.pallas import tpu_sc as plsc

N = 10000
D = 128
N_PAD = 10240
E = 320000
NC = 2
NS = 16
NW = NC * NS
CH = 128
K = 2 * (-(-E // (NW * CH * 2)))
E_PAD = NW * CH * K
NCHUNK = E_PAD // CH
KA = 80
KB = NCHUNK // NS - KA
HB = 16
ROWS_PER_TILE = N_PAD // NS
BT = 512

_mesh = plsc.VectorSubcoreMesh(core_axis_name="c", subcore_axis_name="s")


@functools.partial(
    pl.kernel,
    out_type=(
        jax.ShapeDtypeStruct((NC, N_PAD, D), jnp.float32),
        jax.ShapeDtypeStruct((NC, N_PAD, D), jnp.float32),
    ),
    mesh=_mesh,
    scratch_types=[
        pltpu.VMEM((K, CH), jnp.int32),
        pltpu.VMEM((CH, D), jnp.float32),
        pltpu.VMEM((CH, D), jnp.float32),
        pltpu.VMEM_SHARED((N_PAD, D), jnp.float32),
    ],
)
def _sc_degrees(ones_hbm, src_hbm, dst_hbm, out_s, out_d, idx_v, ones_v,
                buf_v, deg_sh):
    c = lax.axis_index("c")
    s = lax.axis_index("s")
    pltpu.sync_copy(ones_hbm.at[0], ones_v)
    w = s * NC + c

    for idx_hbm, out_hbm in ((src_hbm, out_s), (dst_hbm, out_d)):
        pltpu.sync_copy(ones_hbm.at[1], buf_v)
        pltpu.sync_copy(idx_hbm.at[pl.ds(w * K, K)], idx_v)
        for t in range(ROWS_PER_TILE // CH):
            r0 = s * ROWS_PER_TILE + t * CH
            pltpu.sync_copy(buf_v, deg_sh.at[pl.ds(r0, CH)])
        plsc.subcore_barrier()

        def body(j, carry):
            pltpu.sync_copy(ones_v, deg_sh.at[idx_v.at[j]], add=True)
            return carry

        lax.fori_loop(0, K, body, 0)
        plsc.subcore_barrier()
        for t in range(ROWS_PER_TILE // CH):
            r0 = s * ROWS_PER_TILE + t * CH
            pltpu.sync_copy(deg_sh.at[pl.ds(r0, CH)], buf_v)
            pltpu.sync_copy(buf_v, out_hbm.at[c, pl.ds(r0, CH)])
        plsc.subcore_barrier()


@functools.partial(
    pl.kernel,
    out_type=jax.ShapeDtypeStruct((NC, N_PAD, D), jnp.float32),
    mesh=_mesh,
    scratch_types=[
        pltpu.VMEM((HB, CH), jnp.int32),
        pltpu.VMEM((HB, CH), jnp.int32),
        pltpu.VMEM((CH, D), jnp.float32),
        pltpu.VMEM((CH, D), jnp.float32),
        pltpu.VMEM_SHARED((N_PAD, D), jnp.float32),
        pltpu.SemaphoreType.DMA,
        pltpu.SemaphoreType.DMA,
    ],
)
def _sc_aggregate(h_hbm, src_hbm, dst_hbm, out_hbm, src_v, dst_v, rows0,
                  rows1, agg_sh, sem0, sem1):
    c = lax.axis_index("c")
    s = lax.axis_index("s")
    zero16 = jnp.zeros((16,), jnp.float32)

    def zrow(i, carry):
        for j in range(D // 16):
            rows0[i, pl.ds(j * 16, 16)] = zero16
        return carry

    lax.fori_loop(0, CH, zrow, 0)
    for t in range(ROWS_PER_TILE // CH):
        pltpu.sync_copy(rows0, agg_sh.at[pl.ds(s * ROWS_PER_TILE + t * CH, CH)])
    plsc.subcore_barrier()

    start = jnp.where(c == 0, s * KA, NS * KA + s * KB)
    nblocks = jnp.where(c == 0, KA // HB, KB // HB)

    def block(bi, carry):
        b0 = start + bi * HB
        pltpu.sync_copy(src_hbm.at[pl.ds(b0, HB)], src_v)
        pltpu.sync_copy(dst_hbm.at[pl.ds(b0, HB)], dst_v)
        pltpu.async_copy(h_hbm.at[src_v.at[0]], rows0, sem0)

        def body(jj, carry2):
            a = 2 * jj
            pltpu.async_copy(h_hbm.at[src_v.at[a + 1]], rows1, sem1)
            pltpu.make_async_copy(h_hbm.at[src_v.at[a]], rows0, sem0).wait()
            pltpu.sync_copy(rows0, agg_sh.at[dst_v.at[a]], add=True)

            @pl.when(jj < HB // 2 - 1)
            def _():
                pltpu.async_copy(h_hbm.at[src_v.at[a + 2]], rows0, sem0)

            pltpu.make_async_copy(h_hbm.at[src_v.at[a + 1]], rows1, sem1).wait()
            pltpu.sync_copy(rows1, agg_sh.at[dst_v.at[a + 1]], add=True)
            return carry2

        lax.fori_loop(0, HB // 2, body, 0)
        return carry

    lax.fori_loop(0, nblocks, block, 0)
    plsc.subcore_barrier()
    for t in range(ROWS_PER_TILE // CH):
        r0 = s * ROWS_PER_TILE + t * CH
        pltpu.sync_copy(agg_sh.at[pl.ds(r0, CH)], rows0)
        pltpu.sync_copy(rows0, out_hbm.at[c, pl.ds(r0, CH)])


def _tc_pre_body(deg_s_ref, deg_d_ref, x_ref, rsout_ref, rsin_ref, h0_ref):
    ds_ = deg_s_ref[0] + deg_s_ref[1]
    dd_ = deg_d_ref[0] + deg_d_ref[1]
    so = lax.rsqrt(jnp.maximum(ds_[:, :1], 1.0))
    si = lax.rsqrt(jnp.maximum(dd_[:, :1], 1.0))
    rsout = jnp.broadcast_to(so, (BT, D))
    rsin = jnp.broadcast_to(si, (BT, D))
    rsout_ref[...] = rsout
    rsin_ref[...] = rsin
    h0_ref[...] = x_ref[...] * rsout


def _tc_pre(deg_s, deg_d, x):
    grid = (N_PAD // BT,)
    return pl.pallas_call(
        _tc_pre_body,
        grid=grid,
        in_specs=[
            pl.BlockSpec((NC, BT, D), lambda i: (0, i, 0)),
            pl.BlockSpec((NC, BT, D), lambda i: (0, i, 0)),
            pl.BlockSpec((BT, D), lambda i: (i, 0)),
        ],
        out_specs=[
            pl.BlockSpec((BT, D), lambda i: (i, 0)),
            pl.BlockSpec((BT, D), lambda i: (i, 0)),
            pl.BlockSpec((BT, D), lambda i: (i, 0)),
        ],
        out_shape=[
            jax.ShapeDtypeStruct((N_PAD, D), jnp.float32),
            jax.ShapeDtypeStruct((N_PAD, D), jnp.float32),
            jax.ShapeDtypeStruct((N_PAD, D), jnp.float32),
        ],
    )(deg_s, deg_d, x)


def _tc_mid_body(agg_ref, rsin_ref, rsout_ref, w_ref, b_ref, out_ref):
    a = (agg_ref[0] + agg_ref[1]) * rsin_ref[...]
    h = jnp.dot(a, w_ref[...], preferred_element_type=jnp.float32) + b_ref[...]
    out_ref[...] = jnp.maximum(h, 0.0) * rsout_ref[...]


def _tc_mid(agg, rsin, rsout, w, b):
    grid = (N_PAD // BT,)
    return pl.pallas_call(
        _tc_mid_body,
        grid=grid,
        in_specs=[
            pl.BlockSpec((NC, BT, D), lambda i: (0, i, 0)),
            pl.BlockSpec((BT, D), lambda i: (i, 0)),
            pl.BlockSpec((BT, D), lambda i: (i, 0)),
            pl.BlockSpec((D, D), lambda i: (0, 0)),
            pl.BlockSpec((1, D), lambda i: (0, 0)),
        ],
        out_specs=pl.BlockSpec((BT, D), lambda i: (i, 0)),
        out_shape=jax.ShapeDtypeStruct((N_PAD, D), jnp.float32),
    )(agg, rsin, rsout, w, b)


def _tc_fin_body(agg_ref, rsin_ref, w_ref, b_ref, out_ref):
    i = pl.program_id(0)
    a = (agg_ref[0] + agg_ref[1]) * rsin_ref[...]
    h = jnp.dot(a, w_ref[...], preferred_element_type=jnp.float32) + b_ref[...]
    h = jnp.maximum(h, 0.0)
    rows = lax.broadcasted_iota(jnp.int32, (BT, D), 0) + i * BT
    h = jnp.where(rows < N, h, 0.0)
    part = jnp.sum(h, axis=0, keepdims=True) * (1.0 / N)

    @pl.when(i == 0)
    def _():
        out_ref[...] = part

    @pl.when(i > 0)
    def _():
        out_ref[...] = out_ref[...] + part


def _tc_fin(agg, rsin, w, b):
    grid = (N_PAD // BT,)
    return pl.pallas_call(
        _tc_fin_body,
        grid=grid,
        in_specs=[
            pl.BlockSpec((NC, BT, D), lambda i: (0, i, 0)),
            pl.BlockSpec((BT, D), lambda i: (i, 0)),
            pl.BlockSpec((D, D), lambda i: (0, 0)),
            pl.BlockSpec((1, D), lambda i: (0, 0)),
        ],
        out_specs=pl.BlockSpec((1, D), lambda i: (0, 0)),
        out_shape=jax.ShapeDtypeStruct((1, D), jnp.float32),
    )(agg, rsin, w, b)


def kernel(n_feat, edge_index, W1, b1, W2, b2, W3, b3):
    src = edge_index[0].astype(jnp.int32)
    dst = edge_index[1].astype(jnp.int32)
    npad_e = E_PAD - E
    pad_iota = jnp.arange(npad_e, dtype=jnp.int32)
    pad_src = N + pad_iota % (N_PAD - N)
    pad_dst = N + (pad_iota + 128) % (N_PAD - N)
    src_p = jnp.concatenate([src, pad_src]).reshape(NCHUNK, CH)
    dst_p = jnp.concatenate([dst, pad_dst]).reshape(NCHUNK, CH)
    x_p = jnp.pad(n_feat, ((0, N_PAD - N), (0, 0)))

    ones_c = jnp.stack([jnp.ones((CH, D), jnp.float32),
                        jnp.zeros((CH, D), jnp.float32)])
    deg_s, deg_d = _sc_degrees(ones_c, src_p, dst_p)
    rsout, rsin, h = _tc_pre(deg_s, deg_d, x_p)
    for w, b in ((W1, b1), (W2, b2)):
        agg = _sc_aggregate(h, src_p, dst_p)
        h = _tc_mid(agg, rsin, rsout, w, b.reshape(1, D))
    agg = _sc_aggregate(h, src_p, dst_p)
    return _tc_fin(agg, rsin, W3, b3.reshape(1, D))
